# baseline (speedup 1.0000x reference)
import jax
import jax.numpy as jnp
from jax.experimental import pallas as pl


def _rowstats_body(T, L, RC, x_ref, max_ref, arg_ref):
    iota = jax.lax.broadcasted_iota(jnp.int32, (1, T), 1).astype(jnp.float32)
    for i in range(L // RC):
        xr = x_ref[0, pl.ds(i * RC, RC), :]
        rmax = jnp.max(xr, axis=-1)
        cand = jnp.where(xr == rmax[:, None], iota, float(T))
        rarg = jnp.min(cand, axis=-1).astype(jnp.int32)
        max_ref[0, 0, pl.ds(i * RC, RC)] = rmax
        arg_ref[0, 0, pl.ds(i * RC, RC)] = rarg


def _bincount_body(T, L, RC, a_ref, out_ref):
    hist = jnp.zeros((T,), jnp.int32)
    for i in range(L // RC):
        ar = a_ref[0, pl.ds(i * RC, RC)]
        iota = jax.lax.broadcasted_iota(jnp.int32, (RC, T), 1)
        hist = hist + jnp.sum((iota == ar[:, None]).astype(jnp.int32), axis=0)
    out_ref[0, :] = hist


def kernel(att_ws):
    n_layers, n_heads, L, T = att_ws.shape
    H = n_layers * n_heads
    flat = att_ws.reshape(H, L, T)
    RC = min(512, L)
    RCB = L

    row_max, row_arg = pl.pallas_call(
        lambda x_ref, max_ref, arg_ref: _rowstats_body(T, L // 2, RC, x_ref, max_ref, arg_ref),
        grid=(H, 2),
        in_specs=[pl.BlockSpec((1, L // 2, T), lambda h, r: (h, r, 0))],
        out_specs=[
            pl.BlockSpec((1, 1, L // 2), lambda h, r: (h, 0, r)),
            pl.BlockSpec((1, 1, L // 2), lambda h, r: (h, 0, r)),
        ],
        out_shape=[
            jax.ShapeDtypeStruct((H, 1, L), jnp.float32),
            jax.ShapeDtypeStruct((H, 1, L), jnp.int32),
        ],
    )(flat)

    row_max = row_max[:, 0, :]
    scores = jnp.mean(row_max, axis=-1)
    focus_rate = jnp.max(scores)
    best = jnp.argmax(scores)
    argmax_t = row_arg[best]

    durations = pl.pallas_call(
        lambda a_ref, out_ref: _bincount_body(T, L, RCB, a_ref, out_ref),
        in_specs=[pl.BlockSpec((1, L), lambda: (0, 0))],
        out_specs=pl.BlockSpec((1, T), lambda: (0, 0)),
        out_shape=jax.ShapeDtypeStruct((1, T), jnp.int32),
    )(argmax_t)[0]

    return durations, focus_rate


# speedup vs baseline: 1.1106x; 1.1106x over previous
import jax
import jax.numpy as jnp
from jax.experimental import pallas as pl


def _rowstats_body(T, L, RC, x_ref, max_ref, arg_ref):
    iota = jax.lax.broadcasted_iota(jnp.int32, (1, T), 1).astype(jnp.float32)
    for i in range(L // RC):
        xr = x_ref[0, pl.ds(i * RC, RC), :]
        rmax = jnp.max(xr, axis=-1)
        cand = jnp.where(xr == rmax[:, None], iota, float(T))
        rarg = jnp.min(cand, axis=-1).astype(jnp.int32)
        max_ref[0, 0, pl.ds(i * RC, RC)] = rmax
        arg_ref[0, 0, pl.ds(i * RC, RC)] = rarg


def _bincount_body(T, L, RC, a_ref, out_ref):
    hist = jnp.zeros((T,), jnp.int32)
    for i in range(L // RC):
        ar = a_ref[0, pl.ds(i * RC, RC)]
        iota = jax.lax.broadcasted_iota(jnp.int32, (RC, T), 1)
        hist = hist + jnp.sum((iota == ar[:, None]).astype(jnp.int32), axis=0)
    out_ref[0, :] = hist


def kernel(att_ws):
    n_layers, n_heads, L, T = att_ws.shape
    H = n_layers * n_heads
    flat = att_ws.reshape(H, L, T)
    RC = min(512, L)
    RCB = L

    row_max, row_arg = pl.pallas_call(
        lambda x_ref, max_ref, arg_ref: _rowstats_body(T, L, RC, x_ref, max_ref, arg_ref),
        grid=(H,),
        in_specs=[pl.BlockSpec((1, L, T), lambda h: (h, 0, 0))],
        out_specs=[
            pl.BlockSpec((1, 1, L), lambda h: (h, 0, 0)),
            pl.BlockSpec((1, 1, L), lambda h: (h, 0, 0)),
        ],
        out_shape=[
            jax.ShapeDtypeStruct((H, 1, L), jnp.float32),
            jax.ShapeDtypeStruct((H, 1, L), jnp.int32),
        ],
    )(flat)

    row_max = row_max[:, 0, :]
    scores = jnp.mean(row_max, axis=-1)
    focus_rate = jnp.max(scores)
    best = jnp.argmax(scores)
    argmax_t = row_arg[best]

    durations = pl.pallas_call(
        lambda a_ref, out_ref: _bincount_body(T, L, RCB, a_ref, out_ref),
        in_specs=[pl.BlockSpec((1, L), lambda: (0, 0))],
        out_specs=pl.BlockSpec((1, T), lambda: (0, 0)),
        out_shape=jax.ShapeDtypeStruct((1, T), jnp.int32),
    )(argmax_t)[0]

    return durations, focus_rate


# speedup vs baseline: 1.1231x; 1.0112x over previous
import jax
import jax.numpy as jnp
from jax.experimental import pallas as pl
from jax.experimental.pallas import tpu as pltpu


def _rowstats_body(T, L, RC, x_ref, max_ref, arg_ref):
    iota = jax.lax.broadcasted_iota(jnp.int32, (1, T), 1).astype(jnp.float32)
    for i in range(L // RC):
        xr = x_ref[0, pl.ds(i * RC, RC), :]
        rmax = jnp.max(xr, axis=-1)
        cand = jnp.where(xr == rmax[:, None], iota, float(T))
        rarg = jnp.min(cand, axis=-1).astype(jnp.int32)
        max_ref[0, 0, pl.ds(i * RC, RC)] = rmax
        arg_ref[0, 0, pl.ds(i * RC, RC)] = rarg


def _bincount_body(T, L, RC, best_ref, a_ref, out_ref):
    hist = jnp.zeros((T,), jnp.int32)
    for i in range(L // RC):
        ar = a_ref[0, 0, pl.ds(i * RC, RC)]
        iota = jax.lax.broadcasted_iota(jnp.int32, (RC, T), 1)
        hist = hist + jnp.sum((iota == ar[:, None]).astype(jnp.int32), axis=0)
    out_ref[0, :] = hist


def kernel(att_ws):
    n_layers, n_heads, L, T = att_ws.shape
    H = n_layers * n_heads
    flat = att_ws.reshape(H, L, T)
    RC = min(512, L)
    RCB = L

    row_max, row_arg = pl.pallas_call(
        lambda x_ref, max_ref, arg_ref: _rowstats_body(T, L, RC, x_ref, max_ref, arg_ref),
        grid=(H,),
        in_specs=[pl.BlockSpec((1, L, T), lambda h: (h, 0, 0))],
        out_specs=[
            pl.BlockSpec((1, 1, L), lambda h: (h, 0, 0)),
            pl.BlockSpec((1, 1, L), lambda h: (h, 0, 0)),
        ],
        out_shape=[
            jax.ShapeDtypeStruct((H, 1, L), jnp.float32),
            jax.ShapeDtypeStruct((H, 1, L), jnp.int32),
        ],
    )(flat)

    row_max = row_max[:, 0, :]
    scores = jnp.mean(row_max, axis=-1)
    focus_rate = jnp.max(scores)
    best = jnp.argmax(scores)

    durations = pl.pallas_call(
        lambda best_ref, a_ref, out_ref: _bincount_body(T, L, RCB, best_ref, a_ref, out_ref),
        grid_spec=pltpu.PrefetchScalarGridSpec(
            num_scalar_prefetch=1,
            grid=(1,),
            in_specs=[pl.BlockSpec((1, 1, L), lambda i, best_ref: (best_ref[0], 0, 0))],
            out_specs=pl.BlockSpec((1, T), lambda i, best_ref: (0, 0)),
        ),
        out_shape=jax.ShapeDtypeStruct((1, T), jnp.int32),
    )(best.reshape(1), row_arg)[0]

    return durations, focus_rate
